# Initial kernel scaffold; baseline (speedup 1.0000x reference)
import functools

import jax
import jax.numpy as jnp
from jax import lax
from jax.experimental import pallas as pl
from jax.experimental.pallas import tpu as pltpu
from jax.experimental.pallas import tpu_sc as plsc

N = 100000
E = 800000
NSUB = 16
E_PAD = 819200
EPT = E_PAD // NSUB
CH = 2048
ROWS_PER_CHUNK = CH // 128
N_CHUNKS = EPT // CH
ROWS_PER_TILE = N // NSUB

_MESH = plsc.VectorSubcoreMesh(
    core_axis_name="c", subcore_axis_name="s", num_cores=2, num_subcores=NSUB
)


def _edge_stream(src_e, dst_e, table, acc, idx_s, idx_d, rows, sem_g, sem_s, s):
    def chunk(i, carry):
        row0 = s * (EPT // 128) + i * ROWS_PER_CHUNK
        pltpu.sync_copy(src_e.at[pl.ds(row0, ROWS_PER_CHUNK)], idx_s)
        pltpu.sync_copy(dst_e.at[pl.ds(row0, ROWS_PER_CHUNK)], idx_d)
        gathers = [
            pltpu.async_copy(
                table.at[idx_s.at[j]], rows.at[pl.ds(j * 128, 128)], sem_g
            )
            for j in range(ROWS_PER_CHUNK)
        ]
        for d in gathers:
            d.wait()
        scatters = [
            pltpu.async_copy(
                rows.at[pl.ds(j * 128, 128)], acc.at[idx_d.at[j]], sem_s, add=True
            )
            for j in range(ROWS_PER_CHUNK)
        ]
        for d in scatters:
            d.wait()
        return carry

    lax.fori_loop(0, N_CHUNKS, chunk, 0)


def _zero_acc(z_hbm, acc, s):
    r0 = s * ROWS_PER_TILE
    pltpu.sync_copy(z_hbm.at[pl.ds(r0, ROWS_PER_TILE)], acc.at[pl.ds(r0, ROWS_PER_TILE)])


def _writeout(acc, out_hbm, s):
    r0 = s * ROWS_PER_TILE
    pltpu.sync_copy(acc.at[pl.ds(r0, ROWS_PER_TILE)], out_hbm.at[pl.ds(r0, ROWS_PER_TILE)])


def _sc_agg1_body(xp, src_t, dst_t, src_i, dst_i, z8,
                  aggt, aggi,
                  idx_s, idx_d, rows, acc, sem_g, sem_s):
    c = lax.axis_index("c")
    s = lax.axis_index("s")
    _zero_acc(z8, acc, s)
    plsc.subcore_barrier()

    @pl.when(c == 0)
    def _():
        _edge_stream(src_t, dst_t, xp, acc, idx_s, idx_d, rows, sem_g, sem_s, s)
        plsc.subcore_barrier()
        _writeout(acc, aggt, s)

    @pl.when(c == 1)
    def _():
        _edge_stream(src_i, dst_i, xp, acc, idx_s, idx_d, rows, sem_g, sem_s, s)
        plsc.subcore_barrier()
        _writeout(acc, aggi, s)


_sc_agg1 = functools.partial(
    pl.kernel,
    out_type=(
        jax.ShapeDtypeStruct((N, 8), jnp.float32),
        jax.ShapeDtypeStruct((N, 8), jnp.float32),
    ),
    mesh=_MESH,
    scratch_types=[
        pltpu.VMEM((ROWS_PER_CHUNK, 128), jnp.int32),
        pltpu.VMEM((ROWS_PER_CHUNK, 128), jnp.int32),
        pltpu.VMEM((CH, 8), jnp.float32),
        pltpu.VMEM_SHARED((N + 8, 8), jnp.float32),
        pltpu.SemaphoreType.DMA,
        pltpu.SemaphoreType.DMA,
    ],
)(_sc_agg1_body)


def _sc_agg2_body(h1a, h1b, src_t, dst_t, src_i, dst_i, z16,
                  ota, otb, oia, oib,
                  idx_s, idx_d, rows, acc, sem_g, sem_s):
    c = lax.axis_index("c")
    s = lax.axis_index("s")

    def task(h, src_e, dst_e, out_e):
        _zero_acc(z16, acc, s)
        plsc.subcore_barrier()
        _edge_stream(src_e, dst_e, h, acc, idx_s, idx_d, rows, sem_g, sem_s, s)
        plsc.subcore_barrier()
        _writeout(acc, out_e, s)

    @pl.when(c == 0)
    def _():
        task(h1a, src_t, dst_t, ota)
        task(h1a, src_i, dst_i, oia)

    @pl.when(c == 1)
    def _():
        task(h1b, src_t, dst_t, otb)
        task(h1b, src_i, dst_i, oib)


_sc_agg2 = functools.partial(
    pl.kernel,
    out_type=(
        jax.ShapeDtypeStruct((N, 16), jnp.float32),
        jax.ShapeDtypeStruct((N, 16), jnp.float32),
        jax.ShapeDtypeStruct((N, 16), jnp.float32),
        jax.ShapeDtypeStruct((N, 16), jnp.float32),
    ),
    mesh=_MESH,
    scratch_types=[
        pltpu.VMEM((ROWS_PER_CHUNK, 128), jnp.int32),
        pltpu.VMEM((ROWS_PER_CHUNK, 128), jnp.int32),
        pltpu.VMEM((CH, 16), jnp.float32),
        pltpu.VMEM_SHARED((N + 8, 16), jnp.float32),
        pltpu.SemaphoreType.DMA,
        pltpu.SemaphoreType.DMA,
    ],
)(_sc_agg2_body)


BN = 2000


def _tc1_body(aggt_ref, aggi_ref, xp_ref, wt_ref, wi_ref, wr_ref, ws_ref, b_ref,
              h1a_ref, h1b_ref, rcp_ref):
    aggt = aggt_ref[...]
    aggi = aggi_ref[...]
    xp = xp_ref[...]
    cnt = aggi[:, 6:7]
    rcp = 1.0 / jnp.maximum(cnt, 1.0)
    f32 = jnp.float32
    pre = (
        jnp.dot(aggt, wt_ref[...], preferred_element_type=f32)
        + jnp.dot(aggi * rcp, wi_ref[...], preferred_element_type=f32)
        + jnp.dot(xp, wr_ref[...], preferred_element_type=f32)
        + b_ref[...]
    )
    h = jnp.maximum(pre, 0.0) + jnp.dot(xp, ws_ref[...], preferred_element_type=f32)
    h1a_ref[...] = h[:, :16]
    h1b_ref[...] = h[:, 16:32]
    rcp_ref[...] = rcp


def _tc1(aggt, aggi, xp, wt, wi, wr, ws, b):
    grid = N // BN
    blk = lambda shp: pl.BlockSpec(shp, lambda i: (i, 0))
    cst = lambda shp: pl.BlockSpec(shp, lambda i: (0, 0))
    return pl.pallas_call(
        _tc1_body,
        grid=(grid,),
        in_specs=[
            blk((BN, 8)), blk((BN, 8)), blk((BN, 8)),
            cst((8, 32)), cst((8, 32)), cst((8, 32)), cst((8, 32)), cst((1, 32)),
        ],
        out_specs=[blk((BN, 16)), blk((BN, 16)), blk((BN, 1))],
        out_shape=[
            jax.ShapeDtypeStruct((N, 16), jnp.float32),
            jax.ShapeDtypeStruct((N, 16), jnp.float32),
            jax.ShapeDtypeStruct((N, 1), jnp.float32),
        ],
    )(aggt, aggi, xp, wt, wi, wr, ws, b)


def _tc2_body(ta_ref, tb_ref, ia_ref, ib_ref, h1a_ref, h1b_ref, rcp_ref,
              wt_ref, wi_ref, wr_ref, ws_ref, b_ref,
              wd1_ref, bd1_ref, wd2_ref, bd2_ref, out_ref):
    f32 = jnp.float32
    aggt = jnp.concatenate([ta_ref[...], tb_ref[...]], axis=1)
    aggi = jnp.concatenate([ia_ref[...], ib_ref[...]], axis=1) * rcp_ref[...]
    h1 = jnp.concatenate([h1a_ref[...], h1b_ref[...]], axis=1)
    pre = (
        jnp.dot(aggt, wt_ref[...], preferred_element_type=f32)
        + jnp.dot(aggi, wi_ref[...], preferred_element_type=f32)
        + jnp.dot(h1, wr_ref[...], preferred_element_type=f32)
        + b_ref[...]
    )
    h2 = jnp.maximum(pre, 0.0) + jnp.dot(h1, ws_ref[...], preferred_element_type=f32)
    d = jnp.maximum(jnp.dot(h2, wd1_ref[...], preferred_element_type=f32) + bd1_ref[...], 0.0)
    z = jnp.dot(d, wd2_ref[...], preferred_element_type=f32) + bd2_ref[...]
    out_ref[...] = 1.0 / (1.0 + jnp.exp(-z))


def _tc2(ta, tb, ia, ib, h1a, h1b, rcp, wt, wi, wr, ws, b, wd1, bd1, wd2, bd2):
    grid = N // BN
    blk = lambda shp: pl.BlockSpec(shp, lambda i: (i, 0))
    cst = lambda shp: pl.BlockSpec(shp, lambda i: (0, 0))
    return pl.pallas_call(
        _tc2_body,
        grid=(grid,),
        in_specs=[
            blk((BN, 16)), blk((BN, 16)), blk((BN, 16)), blk((BN, 16)),
            blk((BN, 16)), blk((BN, 16)), blk((BN, 1)),
            cst((32, 32)), cst((32, 32)), cst((32, 32)), cst((32, 32)), cst((1, 32)),
            cst((32, 32)), cst((1, 32)), cst((32, 1)), cst((1, 1)),
        ],
        out_specs=[blk((BN, 1))],
        out_shape=[jax.ShapeDtypeStruct((N, 1), jnp.float32)],
    )(ta, tb, ia, ib, h1a, h1b, rcp, wt, wi, wr, ws, b, wd1, bd1, wd2, bd2)[0]


def _pad_edges(idx, fill):
    p = jnp.concatenate([idx, jnp.full((E_PAD - E,), fill, jnp.int32)])
    return p.reshape(E_PAD // 128, 128)


def kernel(x_stroke, ei_temp, ei_int,
           Wt1, Wi1, Wr1, Ws1, b1,
           Wt2, Wi2, Wr2, Ws2, b2,
           Wd1, bd1, Wd2, bd2):
    ones = jnp.ones((N, 1), jnp.float32)
    zeros = jnp.zeros((N, 1), jnp.float32)
    xp = jnp.concatenate([x_stroke, ones, zeros], axis=1)

    src_t = _pad_edges(ei_temp[0], 0)
    dst_t = _pad_edges(ei_temp[1], N)
    src_i = _pad_edges(ei_int[0], 0)
    dst_i = _pad_edges(ei_int[1], N)

    z8 = jnp.zeros((N, 8), jnp.float32)
    z16 = jnp.zeros((N, 16), jnp.float32)

    aggt1, aggi1 = _sc_agg1(xp, src_t, dst_t, src_i, dst_i, z8)

    pad2 = lambda w: jnp.pad(w, ((0, 2), (0, 0)))
    h1a, h1b, rcp = _tc1(
        aggt1, aggi1, xp,
        pad2(Wt1), pad2(Wi1), pad2(Wr1), pad2(Ws1), b1.reshape(1, 32),
    )

    ta, tb, ia, ib = _sc_agg2(h1a, h1b, src_t, dst_t, src_i, dst_i, z16)

    return _tc2(
        ta, tb, ia, ib, h1a, h1b, rcp,
        Wt2, Wi2, Wr2, Ws2, b2.reshape(1, 32),
        Wd1, bd1.reshape(1, 32), Wd2, bd2.reshape(1, 1),
    )


# speedup vs baseline: 9.0405x; 9.0405x over previous
import functools

import jax
import jax.numpy as jnp
from jax import lax
from jax.experimental import pallas as pl
from jax.experimental.pallas import tpu as pltpu
from jax.experimental.pallas import tpu_sc as plsc

N = 100000
E = 800000
NSUB = 16
E_PAD = 819200
EPT = E_PAD // NSUB
CH1 = 2048
CH2 = 1024
N_PAD = 100096
ROWS_PER_TILE = N_PAD // NSUB

_MESH = plsc.VectorSubcoreMesh(
    core_axis_name="c", subcore_axis_name="s", num_cores=2, num_subcores=NSUB
)


def _edge_stream(src_e, dst_e, table, acc, idx_s, idx_d, rows, sem_g, sem_s, s, ch):
    rpc = ch // 128

    def chunk(i, carry):
        row0 = s * (EPT // 128) + i * rpc
        pltpu.sync_copy(src_e.at[pl.ds(row0, rpc)], idx_s)
        pltpu.sync_copy(dst_e.at[pl.ds(row0, rpc)], idx_d)
        gathers = [
            pltpu.async_copy(
                table.at[idx_s.at[j]], rows.at[pl.ds(j * 128, 128)], sem_g
            )
            for j in range(rpc)
        ]
        for d in gathers:
            d.wait()
        scatters = [
            pltpu.async_copy(
                rows.at[pl.ds(j * 128, 128)], acc.at[idx_d.at[j]], sem_s, add=True
            )
            for j in range(rpc)
        ]
        for d in scatters:
            d.wait()
        return carry

    lax.fori_loop(0, EPT // ch, chunk, 0)


def _zero_acc(z_hbm, acc, s):
    r0 = s * ROWS_PER_TILE
    pltpu.sync_copy(z_hbm.at[pl.ds(r0, ROWS_PER_TILE)], acc.at[pl.ds(r0, ROWS_PER_TILE)])


def _writeout(acc, out_hbm, s):
    r0 = s * ROWS_PER_TILE
    pltpu.sync_copy(acc.at[pl.ds(r0, ROWS_PER_TILE)], out_hbm.at[pl.ds(r0, ROWS_PER_TILE)])


def _sc_agg1_body(xp, src_t, dst_t, src_i, dst_i, z8,
                  aggt, aggi,
                  idx_s, idx_d, rows, acc, sem_g, sem_s):
    c = lax.axis_index("c")
    s = lax.axis_index("s")
    _zero_acc(z8, acc, s)
    plsc.subcore_barrier()

    @pl.when(c == 0)
    def _():
        _edge_stream(src_t, dst_t, xp, acc, idx_s, idx_d, rows, sem_g, sem_s, s, CH1)
        plsc.subcore_barrier()
        _writeout(acc, aggt, s)

    @pl.when(c == 1)
    def _():
        _edge_stream(src_i, dst_i, xp, acc, idx_s, idx_d, rows, sem_g, sem_s, s, CH1)
        plsc.subcore_barrier()
        _writeout(acc, aggi, s)


_SC_PARAMS = pltpu.CompilerParams(use_tc_tiling_on_sc=False, internal_scratch_in_bytes=131072)

_sc_agg1 = functools.partial(
    pl.kernel,
    compiler_params=_SC_PARAMS,
    out_type=(
        jax.ShapeDtypeStruct((N_PAD, 8), jnp.float32),
        jax.ShapeDtypeStruct((N_PAD, 8), jnp.float32),
    ),
    mesh=_MESH,
    scratch_types=[
        pltpu.VMEM((CH1 // 128, 128), jnp.int32),
        pltpu.VMEM((CH1 // 128, 128), jnp.int32),
        pltpu.VMEM((CH1, 8), jnp.float32),
        pltpu.VMEM_SHARED((N_PAD, 8), jnp.float32),
        pltpu.SemaphoreType.DMA,
        pltpu.SemaphoreType.DMA,
    ],
)(_sc_agg1_body)


def _sc_agg2_body(h1a, h1b, src_t, dst_t, src_i, dst_i, z16,
                  ota, otb, oia, oib,
                  idx_s, idx_d, rows, acc, sem_g, sem_s):
    c = lax.axis_index("c")
    s = lax.axis_index("s")

    def task(h, src_e, dst_e, out_e):
        _zero_acc(z16, acc, s)
        plsc.subcore_barrier()
        _edge_stream(src_e, dst_e, h, acc, idx_s, idx_d, rows, sem_g, sem_s, s, CH2)
        plsc.subcore_barrier()
        _writeout(acc, out_e, s)

    @pl.when(c == 0)
    def _():
        task(h1a, src_t, dst_t, ota)
        task(h1a, src_i, dst_i, oia)

    @pl.when(c == 1)
    def _():
        task(h1b, src_t, dst_t, otb)
        task(h1b, src_i, dst_i, oib)


_sc_agg2 = functools.partial(
    pl.kernel,
    compiler_params=_SC_PARAMS,
    out_type=(
        jax.ShapeDtypeStruct((N_PAD, 16), jnp.float32),
        jax.ShapeDtypeStruct((N_PAD, 16), jnp.float32),
        jax.ShapeDtypeStruct((N_PAD, 16), jnp.float32),
        jax.ShapeDtypeStruct((N_PAD, 16), jnp.float32),
    ),
    mesh=_MESH,
    scratch_types=[
        pltpu.VMEM((CH2 // 128, 128), jnp.int32),
        pltpu.VMEM((CH2 // 128, 128), jnp.int32),
        pltpu.VMEM((CH2, 16), jnp.float32),
        pltpu.VMEM_SHARED((N_PAD, 16), jnp.float32),
        pltpu.SemaphoreType.DMA,
        pltpu.SemaphoreType.DMA,
    ],
)(_sc_agg2_body)


BN = 2000


def _tc1_body(aggt_ref, aggi_ref, xp_ref, wt_ref, wi_ref, wr_ref, ws_ref, b_ref,
              h1a_ref, h1b_ref, rcp_ref):
    aggt = aggt_ref[...]
    aggi = aggi_ref[...]
    xp = xp_ref[...]
    cnt = aggi[:, 6:7]
    rcp = 1.0 / jnp.maximum(cnt, 1.0)
    f32 = jnp.float32
    pre = (
        jnp.dot(aggt, wt_ref[...], preferred_element_type=f32)
        + jnp.dot(aggi * rcp, wi_ref[...], preferred_element_type=f32)
        + jnp.dot(xp, wr_ref[...], preferred_element_type=f32)
        + b_ref[...]
    )
    h = jnp.maximum(pre, 0.0) + jnp.dot(xp, ws_ref[...], preferred_element_type=f32)
    h1a_ref[...] = h[:, :16]
    h1b_ref[...] = h[:, 16:32]
    rcp_ref[...] = rcp


def _tc1(aggt, aggi, xp, wt, wi, wr, ws, b):
    grid = N // BN
    blk = lambda shp: pl.BlockSpec(shp, lambda i: (i, 0))
    cst = lambda shp: pl.BlockSpec(shp, lambda i: (0, 0))
    return pl.pallas_call(
        _tc1_body,
        grid=(grid,),
        in_specs=[
            blk((BN, 8)), blk((BN, 8)), blk((BN, 8)),
            cst((8, 32)), cst((8, 32)), cst((8, 32)), cst((8, 32)), cst((1, 32)),
        ],
        out_specs=[blk((BN, 16)), blk((BN, 16)), blk((BN, 1))],
        out_shape=[
            jax.ShapeDtypeStruct((N, 16), jnp.float32),
            jax.ShapeDtypeStruct((N, 16), jnp.float32),
            jax.ShapeDtypeStruct((N, 1), jnp.float32),
        ],
    )(aggt, aggi, xp, wt, wi, wr, ws, b)


def _tc2_body(ta_ref, tb_ref, ia_ref, ib_ref, h1a_ref, h1b_ref, rcp_ref,
              wt_ref, wi_ref, wr_ref, ws_ref, b_ref,
              wd1_ref, bd1_ref, wd2_ref, bd2_ref, out_ref):
    f32 = jnp.float32
    aggt = jnp.concatenate([ta_ref[...], tb_ref[...]], axis=1)
    aggi = jnp.concatenate([ia_ref[...], ib_ref[...]], axis=1) * rcp_ref[...]
    h1 = jnp.concatenate([h1a_ref[...], h1b_ref[...]], axis=1)
    pre = (
        jnp.dot(aggt, wt_ref[...], preferred_element_type=f32)
        + jnp.dot(aggi, wi_ref[...], preferred_element_type=f32)
        + jnp.dot(h1, wr_ref[...], preferred_element_type=f32)
        + b_ref[...]
    )
    h2 = jnp.maximum(pre, 0.0) + jnp.dot(h1, ws_ref[...], preferred_element_type=f32)
    d = jnp.maximum(jnp.dot(h2, wd1_ref[...], preferred_element_type=f32) + bd1_ref[...], 0.0)
    z = jnp.dot(d, wd2_ref[...], preferred_element_type=f32) + bd2_ref[...]
    out_ref[...] = 1.0 / (1.0 + jnp.exp(-z))


def _tc2(ta, tb, ia, ib, h1a, h1b, rcp, wt, wi, wr, ws, b, wd1, bd1, wd2, bd2):
    grid = N // BN
    blk = lambda shp: pl.BlockSpec(shp, lambda i: (i, 0))
    cst = lambda shp: pl.BlockSpec(shp, lambda i: (0, 0))
    return pl.pallas_call(
        _tc2_body,
        grid=(grid,),
        in_specs=[
            blk((BN, 16)), blk((BN, 16)), blk((BN, 16)), blk((BN, 16)),
            blk((BN, 16)), blk((BN, 16)), blk((BN, 1)),
            cst((32, 32)), cst((32, 32)), cst((32, 32)), cst((32, 32)), cst((1, 32)),
            cst((32, 32)), cst((1, 32)), cst((32, 1)), cst((1, 1)),
        ],
        out_specs=[blk((BN, 1))],
        out_shape=[jax.ShapeDtypeStruct((N, 1), jnp.float32)],
    )(ta, tb, ia, ib, h1a, h1b, rcp, wt, wi, wr, ws, b, wd1, bd1, wd2, bd2)[0]


def _pad_edges(idx, fill):
    p = jnp.concatenate([idx, jnp.full((E_PAD - E,), fill, jnp.int32)])
    return p.reshape(E_PAD // 128, 128)


def kernel(x_stroke, ei_temp, ei_int,
           Wt1, Wi1, Wr1, Ws1, b1,
           Wt2, Wi2, Wr2, Ws2, b2,
           Wd1, bd1, Wd2, bd2):
    ones = jnp.ones((N, 1), jnp.float32)
    zeros = jnp.zeros((N, 1), jnp.float32)
    xp = jnp.concatenate([x_stroke, ones, zeros], axis=1)

    src_t = _pad_edges(ei_temp[0], 0)
    dst_t = _pad_edges(ei_temp[1], N)
    src_i = _pad_edges(ei_int[0], 0)
    dst_i = _pad_edges(ei_int[1], N)

    z8 = jnp.zeros((N_PAD, 8), jnp.float32)
    z16 = jnp.zeros((N_PAD, 16), jnp.float32)

    aggt1, aggi1 = _sc_agg1(xp, src_t, dst_t, src_i, dst_i, z8)

    pad2 = lambda w: jnp.pad(w, ((0, 2), (0, 0)))
    h1a, h1b, rcp = _tc1(
        aggt1, aggi1, xp,
        pad2(Wt1), pad2(Wi1), pad2(Wr1), pad2(Ws1), b1.reshape(1, 32),
    )

    ta, tb, ia, ib = _sc_agg2(h1a, h1b, src_t, dst_t, src_i, dst_i, z16)

    return _tc2(
        ta, tb, ia, ib, h1a, h1b, rcp,
        Wt2, Wi2, Wr2, Ws2, b2.reshape(1, 32),
        Wd1, bd1.reshape(1, 32), Wd2, bd2.reshape(1, 1),
    )


# speedup vs baseline: 10.0107x; 1.1073x over previous
import functools

import jax
import jax.numpy as jnp
from jax import lax
from jax.experimental import pallas as pl
from jax.experimental.pallas import tpu as pltpu
from jax.experimental.pallas import tpu_sc as plsc

N = 100000
E = 800000
NSUB = 16
E_PAD = 819200
EPT = E_PAD // NSUB
CH = 512
RPC = CH // 128
NCH = EPT // CH
N_PAD = 100096
ROWS_PER_TILE = N_PAD // NSUB

_MESH = plsc.VectorSubcoreMesh(
    core_axis_name="c", subcore_axis_name="s", num_cores=2, num_subcores=NSUB
)

_SC_PARAMS = pltpu.CompilerParams(use_tc_tiling_on_sc=False)


def _edge_stream(packed, table, acc, idx, rows, sem_g, sem_s, s):
    D = table.shape[1]

    def load_fire(i, b):
        row0 = (s * NCH + i) * (2 * RPC)
        pltpu.sync_copy(packed.at[pl.ds(row0, 2 * RPC)], idx[b])
        for j in range(RPC):
            pltpu.async_copy(
                table.at[idx[b].at[j]], rows[b].at[pl.ds(j * 128, 128)], sem_g[b]
            )

    def fire_scatters(b):
        for j in range(RPC):
            pltpu.async_copy(
                rows[b].at[pl.ds(j * 128, 128)], acc.at[idx[b].at[RPC + j]],
                sem_s[b], add=True,
            )

    def drain(sem, b):
        pltpu.make_async_copy(table.at[pl.ds(0, CH)], rows[b], sem).wait()

    load_fire(0, 0)
    load_fire(1, 1)
    drain(sem_g[0], 0)
    fire_scatters(0)

    def pair(k, carry):
        drain(sem_s[0], 0)
        load_fire(2 * k, 0)
        drain(sem_g[1], 1)
        fire_scatters(1)
        drain(sem_s[1], 1)
        load_fire(2 * k + 1, 1)
        drain(sem_g[0], 0)
        fire_scatters(0)
        return carry

    lax.fori_loop(1, NCH // 2, pair, 0)
    drain(sem_g[1], 1)
    fire_scatters(1)
    drain(sem_s[0], 0)
    drain(sem_s[1], 1)


def _zero_acc(z_hbm, acc, s):
    r0 = s * ROWS_PER_TILE
    pltpu.sync_copy(z_hbm.at[pl.ds(r0, ROWS_PER_TILE)], acc.at[pl.ds(r0, ROWS_PER_TILE)])


def _writeout(acc, out_hbm, s):
    r0 = s * ROWS_PER_TILE
    pltpu.sync_copy(acc.at[pl.ds(r0, ROWS_PER_TILE)], out_hbm.at[pl.ds(r0, ROWS_PER_TILE)])


def _sc_agg1_body(xp, pk_t, pk_i, z8,
                  aggt, aggi,
                  idx0, idx1, rows0, rows1, acc, sg0, sg1, ss0, ss1):
    c = lax.axis_index("c")
    s = lax.axis_index("s")
    idx = [idx0, idx1]
    rows = [rows0, rows1]
    sem_g = [sg0, sg1]
    sem_s = [ss0, ss1]
    _zero_acc(z8, acc, s)
    plsc.subcore_barrier()

    @pl.when(c == 0)
    def _():
        _edge_stream(pk_t, xp, acc, idx, rows, sem_g, sem_s, s)
        plsc.subcore_barrier()
        _writeout(acc, aggt, s)

    @pl.when(c == 1)
    def _():
        _edge_stream(pk_i, xp, acc, idx, rows, sem_g, sem_s, s)
        plsc.subcore_barrier()
        _writeout(acc, aggi, s)


_sc_agg1 = functools.partial(
    pl.kernel,
    compiler_params=_SC_PARAMS,
    out_type=(
        jax.ShapeDtypeStruct((N_PAD, 8), jnp.float32),
        jax.ShapeDtypeStruct((N_PAD, 8), jnp.float32),
    ),
    mesh=_MESH,
    scratch_types=[
        pltpu.VMEM((2 * RPC, 128), jnp.int32),
        pltpu.VMEM((2 * RPC, 128), jnp.int32),
        pltpu.VMEM((CH, 8), jnp.float32),
        pltpu.VMEM((CH, 8), jnp.float32),
        pltpu.VMEM_SHARED((N_PAD, 8), jnp.float32),
        pltpu.SemaphoreType.DMA,
        pltpu.SemaphoreType.DMA,
        pltpu.SemaphoreType.DMA,
        pltpu.SemaphoreType.DMA,
    ],
)(_sc_agg1_body)


def _sc_agg2_body(h1a, h1b, pk_t, pk_i, z16,
                  ota, otb, oia, oib,
                  idx0, idx1, rows0, rows1, acc, sg0, sg1, ss0, ss1):
    c = lax.axis_index("c")
    s = lax.axis_index("s")
    idx = [idx0, idx1]
    rows = [rows0, rows1]
    sem_g = [sg0, sg1]
    sem_s = [ss0, ss1]

    def task(h, pk, out_e):
        _zero_acc(z16, acc, s)
        plsc.subcore_barrier()
        _edge_stream(pk, h, acc, idx, rows, sem_g, sem_s, s)
        plsc.subcore_barrier()
        _writeout(acc, out_e, s)

    @pl.when(c == 0)
    def _():
        task(h1a, pk_t, ota)
        task(h1a, pk_i, oia)

    @pl.when(c == 1)
    def _():
        task(h1b, pk_t, otb)
        task(h1b, pk_i, oib)


_sc_agg2 = functools.partial(
    pl.kernel,
    compiler_params=_SC_PARAMS,
    out_type=(
        jax.ShapeDtypeStruct((N_PAD, 16), jnp.float32),
        jax.ShapeDtypeStruct((N_PAD, 16), jnp.float32),
        jax.ShapeDtypeStruct((N_PAD, 16), jnp.float32),
        jax.ShapeDtypeStruct((N_PAD, 16), jnp.float32),
    ),
    mesh=_MESH,
    scratch_types=[
        pltpu.VMEM((2 * RPC, 128), jnp.int32),
        pltpu.VMEM((2 * RPC, 128), jnp.int32),
        pltpu.VMEM((CH, 16), jnp.float32),
        pltpu.VMEM((CH, 16), jnp.float32),
        pltpu.VMEM_SHARED((N_PAD, 16), jnp.float32),
        pltpu.SemaphoreType.DMA,
        pltpu.SemaphoreType.DMA,
        pltpu.SemaphoreType.DMA,
        pltpu.SemaphoreType.DMA,
    ],
)(_sc_agg2_body)


BN = 2000


def _tc1_body(aggt_ref, aggi_ref, xp_ref, wt_ref, wi_ref, wr_ref, ws_ref, b_ref,
              h1a_ref, h1b_ref, rcp_ref):
    aggt = aggt_ref[...]
    aggi = aggi_ref[...]
    xp = xp_ref[...]
    cnt = aggi[:, 6:7]
    rcp = 1.0 / jnp.maximum(cnt, 1.0)
    f32 = jnp.float32
    pre = (
        jnp.dot(aggt, wt_ref[...], preferred_element_type=f32)
        + jnp.dot(aggi * rcp, wi_ref[...], preferred_element_type=f32)
        + jnp.dot(xp, wr_ref[...], preferred_element_type=f32)
        + b_ref[...]
    )
    h = jnp.maximum(pre, 0.0) + jnp.dot(xp, ws_ref[...], preferred_element_type=f32)
    h1a_ref[...] = h[:, :16]
    h1b_ref[...] = h[:, 16:32]
    rcp_ref[...] = rcp


def _tc1(aggt, aggi, xp, wt, wi, wr, ws, b):
    grid = N // BN
    blk = lambda shp: pl.BlockSpec(shp, lambda i: (i, 0))
    cst = lambda shp: pl.BlockSpec(shp, lambda i: (0, 0))
    return pl.pallas_call(
        _tc1_body,
        grid=(grid,),
        in_specs=[
            blk((BN, 8)), blk((BN, 8)), blk((BN, 8)),
            cst((8, 32)), cst((8, 32)), cst((8, 32)), cst((8, 32)), cst((1, 32)),
        ],
        out_specs=[blk((BN, 16)), blk((BN, 16)), blk((BN, 1))],
        out_shape=[
            jax.ShapeDtypeStruct((N, 16), jnp.float32),
            jax.ShapeDtypeStruct((N, 16), jnp.float32),
            jax.ShapeDtypeStruct((N, 1), jnp.float32),
        ],
    )(aggt, aggi, xp, wt, wi, wr, ws, b)


def _tc2_body(ta_ref, tb_ref, ia_ref, ib_ref, h1a_ref, h1b_ref, rcp_ref,
              wt_ref, wi_ref, wr_ref, ws_ref, b_ref,
              wd1_ref, bd1_ref, wd2_ref, bd2_ref, out_ref):
    f32 = jnp.float32
    aggt = jnp.concatenate([ta_ref[...], tb_ref[...]], axis=1)
    aggi = jnp.concatenate([ia_ref[...], ib_ref[...]], axis=1) * rcp_ref[...]
    h1 = jnp.concatenate([h1a_ref[...], h1b_ref[...]], axis=1)
    pre = (
        jnp.dot(aggt, wt_ref[...], preferred_element_type=f32)
        + jnp.dot(aggi, wi_ref[...], preferred_element_type=f32)
        + jnp.dot(h1, wr_ref[...], preferred_element_type=f32)
        + b_ref[...]
    )
    h2 = jnp.maximum(pre, 0.0) + jnp.dot(h1, ws_ref[...], preferred_element_type=f32)
    d = jnp.maximum(jnp.dot(h2, wd1_ref[...], preferred_element_type=f32) + bd1_ref[...], 0.0)
    z = jnp.dot(d, wd2_ref[...], preferred_element_type=f32) + bd2_ref[...]
    out_ref[...] = 1.0 / (1.0 + jnp.exp(-z))


def _tc2(ta, tb, ia, ib, h1a, h1b, rcp, wt, wi, wr, ws, b, wd1, bd1, wd2, bd2):
    grid = N // BN
    blk = lambda shp: pl.BlockSpec(shp, lambda i: (i, 0))
    cst = lambda shp: pl.BlockSpec(shp, lambda i: (0, 0))
    return pl.pallas_call(
        _tc2_body,
        grid=(grid,),
        in_specs=[
            blk((BN, 16)), blk((BN, 16)), blk((BN, 16)), blk((BN, 16)),
            blk((BN, 16)), blk((BN, 16)), blk((BN, 1)),
            cst((32, 32)), cst((32, 32)), cst((32, 32)), cst((32, 32)), cst((1, 32)),
            cst((32, 32)), cst((1, 32)), cst((32, 1)), cst((1, 1)),
        ],
        out_specs=[blk((BN, 1))],
        out_shape=[jax.ShapeDtypeStruct((N, 1), jnp.float32)],
    )(ta, tb, ia, ib, h1a, h1b, rcp, wt, wi, wr, ws, b, wd1, bd1, wd2, bd2)[0]


def _pack_edges(ei):
    src_p = jnp.concatenate([ei[0], jnp.zeros((E_PAD - E,), jnp.int32)])
    dst_p = jnp.concatenate([ei[1], jnp.full((E_PAD - E,), N, jnp.int32)])
    s3 = src_p.reshape(E_PAD // CH, RPC, 128)
    d3 = dst_p.reshape(E_PAD // CH, RPC, 128)
    return jnp.concatenate([s3, d3], axis=1).reshape(E_PAD // CH * 2 * RPC, 128)


def kernel(x_stroke, ei_temp, ei_int,
           Wt1, Wi1, Wr1, Ws1, b1,
           Wt2, Wi2, Wr2, Ws2, b2,
           Wd1, bd1, Wd2, bd2):
    ones = jnp.ones((N, 1), jnp.float32)
    zeros = jnp.zeros((N, 1), jnp.float32)
    xp = jnp.concatenate([x_stroke, ones, zeros], axis=1)

    pk_t = _pack_edges(ei_temp)
    pk_i = _pack_edges(ei_int)

    z8 = jnp.zeros((N_PAD, 8), jnp.float32)
    z16 = jnp.zeros((N_PAD, 16), jnp.float32)

    aggt1, aggi1 = _sc_agg1(xp, pk_t, pk_i, z8)

    pad2 = lambda w: jnp.pad(w, ((0, 2), (0, 0)))
    h1a, h1b, rcp = _tc1(
        aggt1, aggi1, xp,
        pad2(Wt1), pad2(Wi1), pad2(Wr1), pad2(Ws1), b1.reshape(1, 32),
    )

    ta, tb, ia, ib = _sc_agg2(h1a, h1b, pk_t, pk_i, z16)

    return _tc2(
        ta, tb, ia, ib, h1a, h1b, rcp,
        Wt2, Wi2, Wr2, Ws2, b2.reshape(1, 32),
        Wd1, bd1.reshape(1, 32), Wd2, bd2.reshape(1, 1),
    )


# speedup vs baseline: 13.9557x; 1.3941x over previous
import functools

import jax
import jax.numpy as jnp
from jax import lax
from jax.experimental import pallas as pl
from jax.experimental.pallas import tpu as pltpu
from jax.experimental.pallas import tpu_sc as plsc

N = 100000
E = 800000
NSUB = 16
E_PAD = 819200
EPT = E_PAD // NSUB
CH = 512
RPC = CH // 128
NCH = EPT // CH
N_PAD = 100096
ROWS_PER_TILE = N_PAD // NSUB

_MESH = plsc.VectorSubcoreMesh(
    core_axis_name="c", subcore_axis_name="s", num_cores=2, num_subcores=NSUB
)

_SC_PARAMS = pltpu.CompilerParams(use_tc_tiling_on_sc=False)


def _edge_stream(packed, table, acc, idx, rows, sem_g, sem_s, s):
    D = table.shape[1]

    def load_fire(i, b):
        row0 = (s * NCH + i) * (2 * RPC)
        pltpu.sync_copy(packed.at[pl.ds(row0, 2 * RPC)], idx[b])
        for j in range(RPC):
            pltpu.async_copy(
                table.at[idx[b].at[j]], rows[b].at[pl.ds(j * 128, 128)], sem_g[b]
            )

    def fire_scatters(b):
        for j in range(RPC):
            pltpu.async_copy(
                rows[b].at[pl.ds(j * 128, 128)], acc.at[idx[b].at[RPC + j]],
                sem_s[b], add=True,
            )

    def drain(sem, b):
        pltpu.make_async_copy(table.at[pl.ds(0, CH)], rows[b], sem).wait()

    load_fire(0, 0)
    load_fire(1, 1)
    drain(sem_g[0], 0)
    fire_scatters(0)

    def pair(k, carry):
        drain(sem_s[0], 0)
        load_fire(2 * k, 0)
        drain(sem_g[1], 1)
        fire_scatters(1)
        drain(sem_s[1], 1)
        load_fire(2 * k + 1, 1)
        drain(sem_g[0], 0)
        fire_scatters(0)
        return carry

    lax.fori_loop(1, NCH // 2, pair, 0)
    drain(sem_g[1], 1)
    fire_scatters(1)
    drain(sem_s[0], 0)
    drain(sem_s[1], 1)


def _zero_acc(z_hbm, acc, s):
    r0 = s * ROWS_PER_TILE
    pltpu.sync_copy(z_hbm.at[pl.ds(r0, ROWS_PER_TILE)], acc.at[pl.ds(r0, ROWS_PER_TILE)])


def _writeout(acc, out_hbm, s):
    r0 = s * ROWS_PER_TILE
    pltpu.sync_copy(acc.at[pl.ds(r0, ROWS_PER_TILE)], out_hbm.at[pl.ds(r0, ROWS_PER_TILE)])


def _sc_agg1_body(xp, pk_t, pk_i, z8,
                  aggt, aggi,
                  idx0, idx1, rows0, rows1, acc, sg0, sg1, ss0, ss1):
    c = lax.axis_index("c")
    s = lax.axis_index("s")
    idx = [idx0, idx1]
    rows = [rows0, rows1]
    sem_g = [sg0, sg1]
    sem_s = [ss0, ss1]
    _zero_acc(z8, acc, s)
    plsc.subcore_barrier()

    @pl.when(c == 0)
    def _():
        _edge_stream(pk_t, xp, acc, idx, rows, sem_g, sem_s, s)
        plsc.subcore_barrier()
        _writeout(acc, aggt, s)

    @pl.when(c == 1)
    def _():
        _edge_stream(pk_i, xp, acc, idx, rows, sem_g, sem_s, s)
        plsc.subcore_barrier()
        _writeout(acc, aggi, s)


_sc_agg1 = functools.partial(
    pl.kernel,
    compiler_params=_SC_PARAMS,
    out_type=(
        jax.ShapeDtypeStruct((N_PAD, 8), jnp.float32),
        jax.ShapeDtypeStruct((N_PAD, 8), jnp.float32),
    ),
    mesh=_MESH,
    scratch_types=[
        pltpu.VMEM((2 * RPC, 128), jnp.int32),
        pltpu.VMEM((2 * RPC, 128), jnp.int32),
        pltpu.VMEM((CH, 8), jnp.float32),
        pltpu.VMEM((CH, 8), jnp.float32),
        pltpu.VMEM_SHARED((N_PAD, 8), jnp.float32),
        pltpu.SemaphoreType.DMA,
        pltpu.SemaphoreType.DMA,
        pltpu.SemaphoreType.DMA,
        pltpu.SemaphoreType.DMA,
    ],
)(_sc_agg1_body)


def _sc_agg2_body(h1a, h1b, pk_t, pk_i, z16,
                  ota, otb, oia, oib,
                  idx0, idx1, rows0, rows1, acc, sg0, sg1, ss0, ss1):
    c = lax.axis_index("c")
    s = lax.axis_index("s")
    idx = [idx0, idx1]
    rows = [rows0, rows1]
    sem_g = [sg0, sg1]
    sem_s = [ss0, ss1]

    def task(h, pk, out_e):
        _zero_acc(z16, acc, s)
        plsc.subcore_barrier()
        _edge_stream(pk, h, acc, idx, rows, sem_g, sem_s, s)
        plsc.subcore_barrier()
        _writeout(acc, out_e, s)

    @pl.when(c == 0)
    def _():
        task(h1a, pk_t, ota)
        task(h1a, pk_i, oia)

    @pl.when(c == 1)
    def _():
        task(h1b, pk_t, otb)
        task(h1b, pk_i, oib)


_sc_agg2 = functools.partial(
    pl.kernel,
    compiler_params=_SC_PARAMS,
    out_type=(
        jax.ShapeDtypeStruct((N_PAD, 16), jnp.float32),
        jax.ShapeDtypeStruct((N_PAD, 16), jnp.float32),
        jax.ShapeDtypeStruct((N_PAD, 16), jnp.float32),
        jax.ShapeDtypeStruct((N_PAD, 16), jnp.float32),
    ),
    mesh=_MESH,
    scratch_types=[
        pltpu.VMEM((2 * RPC, 128), jnp.int32),
        pltpu.VMEM((2 * RPC, 128), jnp.int32),
        pltpu.VMEM((CH, 16), jnp.float32),
        pltpu.VMEM((CH, 16), jnp.float32),
        pltpu.VMEM_SHARED((N_PAD, 16), jnp.float32),
        pltpu.SemaphoreType.DMA,
        pltpu.SemaphoreType.DMA,
        pltpu.SemaphoreType.DMA,
        pltpu.SemaphoreType.DMA,
    ],
)(_sc_agg2_body)


GRID = (N + 2048 - 1) // 2048


def _tc1_body(aggt_ref, aggi_ref, xp_ref,
              wta_ref, wtb_ref, wia_ref, wib_ref, wra_ref, wrb_ref,
              wsa_ref, wsb_ref, ba_ref, bb_ref, ecnt_ref,
              h1a_ref, h1b_ref):
    f32 = jnp.float32
    aggt = aggt_ref[...]
    aggi = aggi_ref[...]
    xp = xp_ref[...]
    dot = lambda a, b: jnp.dot(a, b, preferred_element_type=f32)
    cntb = dot(aggi, ecnt_ref[...])
    aggim = aggi * (1.0 / jnp.maximum(cntb, 1.0))
    preA = dot(aggt, wta_ref[...]) + dot(aggim, wia_ref[...])         + dot(xp, wra_ref[...]) + ba_ref[...]
    h1a_ref[...] = jnp.maximum(preA, 0.0) + dot(xp, wsa_ref[...])
    preB = dot(aggt, wtb_ref[...]) + dot(aggim, wib_ref[...])         + dot(xp, wrb_ref[...]) + bb_ref[...]
    h1b_ref[...] = jnp.maximum(preB, 0.0) + dot(xp, wsb_ref[...])


def _tc1(aggt_p, aggi_p, xp_p, *weights):
    b8 = pl.BlockSpec((128, 128), lambda i: (i, 0))
    b16 = pl.BlockSpec((128, 256), lambda i: (i, 0))
    cst = lambda shp: pl.BlockSpec(shp, lambda i: (0, 0))
    return pl.pallas_call(
        _tc1_body,
        grid=(GRID,),
        in_specs=[
            b8, b8, b8,
            cst((128, 256)), cst((128, 256)), cst((128, 256)), cst((128, 256)),
            cst((128, 256)), cst((128, 256)), cst((128, 256)), cst((128, 256)),
            cst((1, 256)), cst((1, 256)), cst((128, 128)),
        ],
        out_specs=[b16, b16],
        out_shape=[
            jax.ShapeDtypeStruct((N // 16, 256), jnp.float32),
            jax.ShapeDtypeStruct((N // 16, 256), jnp.float32),
        ],
    )(aggt_p, aggi_p, xp_p, *weights)


def _tc2_body(ta_ref, tb_ref, ia_ref, ib_ref, h1a_ref, h1b_ref, aggi1_ref,
              wt_a, wt_b, wi_a, wi_b, wr_a, wr_b, ws_a, ws_b, b2e_ref,
              wd1e_ref, bd1e_ref, wd2e_ref, bd2e_ref, ecnt16_ref, out_ref):
    f32 = jnp.float32
    dot = lambda a, b: jnp.dot(a, b, preferred_element_type=f32)
    cnt16 = dot(aggi1_ref[...], ecnt16_ref[...])
    rcp16 = 1.0 / jnp.maximum(cnt16, 1.0)
    ia = ia_ref[...] * rcp16
    ib = ib_ref[...] * rcp16
    h1a = h1a_ref[...]
    h1b = h1b_ref[...]
    pre = (
        dot(ta_ref[...], wt_a[...]) + dot(tb_ref[...], wt_b[...])
        + dot(ia, wi_a[...]) + dot(ib, wi_b[...])
        + dot(h1a, wr_a[...]) + dot(h1b, wr_b[...]) + b2e_ref[...]
    )
    h2 = jnp.maximum(pre, 0.0) + dot(h1a, ws_a[...]) + dot(h1b, ws_b[...])
    d = jnp.maximum(dot(h2, wd1e_ref[...]) + bd1e_ref[...], 0.0)
    z = dot(d, wd2e_ref[...]) + bd2e_ref[...]
    out_ref[...] = 1.0 / (1.0 + jnp.exp(-z))


def _tc2(ta_p, tb_p, ia_p, ib_p, h1a_p, h1b_p, aggi1_p, *weights):
    b8 = pl.BlockSpec((128, 128), lambda i: (i, 0))
    b16 = pl.BlockSpec((128, 256), lambda i: (i, 0))
    cst = lambda shp: pl.BlockSpec(shp, lambda i: (0, 0))
    return pl.pallas_call(
        _tc2_body,
        grid=(GRID,),
        in_specs=[
            b16, b16, b16, b16, b16, b16, b8,
            cst((256, 512)), cst((256, 512)), cst((256, 512)), cst((256, 512)),
            cst((256, 512)), cst((256, 512)), cst((256, 512)), cst((256, 512)),
            cst((1, 512)),
            cst((512, 512)), cst((1, 512)), cst((512, 16)), cst((1, 16)),
            cst((128, 256)),
        ],
        out_specs=[pl.BlockSpec((128, 16), lambda i: (i, 0))],
        out_shape=[jax.ShapeDtypeStruct((N // 16, 16), jnp.float32)],
    )(ta_p, tb_p, ia_p, ib_p, h1a_p, h1b_p, aggi1_p, *weights)[0]


def _pack_edges(ei):
    src_p = jnp.concatenate([ei[0], jnp.zeros((E_PAD - E,), jnp.int32)])
    dst_p = jnp.concatenate([ei[1], jnp.full((E_PAD - E,), N, jnp.int32)])
    s3 = src_p.reshape(E_PAD // CH, RPC, 128)
    d3 = dst_p.reshape(E_PAD // CH, RPC, 128)
    return jnp.concatenate([s3, d3], axis=1).reshape(E_PAD // CH * 2 * RPC, 128)


def kernel(x_stroke, ei_temp, ei_int,
           Wt1, Wi1, Wr1, Ws1, b1,
           Wt2, Wi2, Wr2, Ws2, b2,
           Wd1, bd1, Wd2, bd2):
    f32 = jnp.float32
    eye16 = jnp.eye(16, dtype=f32)
    kr = lambda w: jnp.kron(eye16, w)

    ones = jnp.ones((N, 1), f32)
    zeros = jnp.zeros((N, 1), f32)
    xp_p = jnp.concatenate([x_stroke, ones, zeros], axis=1).reshape(N // 16, 128)

    pk_t = _pack_edges(ei_temp)
    pk_i = _pack_edges(ei_int)
    z8 = jnp.zeros((N_PAD, 8), f32)
    z16 = jnp.zeros((N_PAD, 16), f32)

    aggt1, aggi1 = _sc_agg1(xp_p.reshape(N, 8), pk_t, pk_i, z8)
    aggt1_p = aggt1.reshape(N_PAD // 16, 128)
    aggi1_p = aggi1.reshape(N_PAD // 16, 128)

    pad2 = lambda w: jnp.pad(w, ((0, 2), (0, 0)))
    Wt1p, Wi1p, Wr1p, Ws1p = pad2(Wt1), pad2(Wi1), pad2(Wr1), pad2(Ws1)
    m8 = jnp.zeros((8, 8), f32).at[6].set(1.0)
    w1 = [kr(Wt1p[:, :16]), kr(Wt1p[:, 16:]), kr(Wi1p[:, :16]), kr(Wi1p[:, 16:]),
          kr(Wr1p[:, :16]), kr(Wr1p[:, 16:]), kr(Ws1p[:, :16]), kr(Ws1p[:, 16:]),
          jnp.tile(b1[:16], 16).reshape(1, 256), jnp.tile(b1[16:], 16).reshape(1, 256),
          kr(m8)]
    h1a_p, h1b_p = _tc1(aggt1_p, aggi1_p, xp_p, *w1)

    ta, tb, ia, ib = _sc_agg2(
        h1a_p.reshape(N, 16), h1b_p.reshape(N, 16), pk_t, pk_i, z16)

    m816 = jnp.zeros((8, 16), f32).at[6].set(1.0)
    w2 = [kr(Wt2[:16]), kr(Wt2[16:]), kr(Wi2[:16]), kr(Wi2[16:]),
          kr(Wr2[:16]), kr(Wr2[16:]), kr(Ws2[:16]), kr(Ws2[16:]),
          jnp.tile(b2, 16).reshape(1, 512),
          kr(Wd1), jnp.tile(bd1, 16).reshape(1, 512),
          kr(Wd2), jnp.tile(bd2, 16).reshape(1, 16),
          kr(m816)]
    out_p = _tc2(
        ta.reshape(N_PAD // 16, 256), tb.reshape(N_PAD // 16, 256),
        ia.reshape(N_PAD // 16, 256), ib.reshape(N_PAD // 16, 256),
        h1a_p, h1b_p, aggi1_p, *w2)
    return out_p.reshape(N, 1)


# speedup vs baseline: 14.6810x; 1.0520x over previous
import functools

import jax
import jax.numpy as jnp
from jax import lax
from jax.experimental import pallas as pl
from jax.experimental.pallas import tpu as pltpu
from jax.experimental.pallas import tpu_sc as plsc

N = 100000
E = 800000
NSUB = 16
E_PAD = 819200
EPT = E_PAD // NSUB
CH1 = 1024
CH2 = 512
N_PAD = 100096
ROWS_PER_TILE = N_PAD // NSUB

_MESH = plsc.VectorSubcoreMesh(
    core_axis_name="c", subcore_axis_name="s", num_cores=2, num_subcores=NSUB
)

_SC_PARAMS = pltpu.CompilerParams(use_tc_tiling_on_sc=False)


def _edge_stream(pk, table, acc, idx, rows, sem_i, sem_g, sem_s, s, ch):
    G = ch // 512
    IR = 8 * G
    nch = EPT // ch

    def load_idx(i, q):
        ic = jnp.minimum(i, nch - 1)
        row0 = (s * nch + ic) * IR
        pltpu.async_copy(pk.at[pl.ds(row0, IR)], idx[q], sem_i[q])

    def wait_idx(q):
        pltpu.make_async_copy(pk.at[pl.ds(0, IR)], idx[q], sem_i[q]).wait()

    def fire_gathers(b, q):
        for g in range(G):
            for j in range(4):
                pltpu.async_copy(
                    table.at[idx[q].at[8 * g + j]],
                    rows[b].at[pl.ds((4 * g + j) * 128, 128)], sem_g[b])

    def fire_scatters(b, q):
        for g in range(G):
            for j in range(4):
                pltpu.async_copy(
                    rows[b].at[pl.ds((4 * g + j) * 128, 128)],
                    acc.at[idx[q].at[8 * g + 4 + j]], sem_s[b], add=True)

    def drain(sem, b):
        pltpu.make_async_copy(table.at[pl.ds(0, ch)], rows[b], sem).wait()

    for q in range(4):
        load_idx(q, q)
    wait_idx(0)
    fire_gathers(0, 0)
    wait_idx(1)
    fire_gathers(1, 1)
    drain(sem_g[0], 0)
    fire_scatters(0, 0)

    def step(i, u):
        b = u % 2
        q = (2 + u) % 4
        drain(sem_s[b], b)
        load_idx(i + 2, u % 4)
        wait_idx(q)
        fire_gathers(b, q)
        drain(sem_g[1 - b], 1 - b)
        fire_scatters(1 - b, (1 + u) % 4)

    n_main = (nch - 2) // 4
    rem = (nch - 2) % 4

    def body(k, carry):
        i0 = 2 + 4 * k
        for u in range(4):
            step(i0 + u, u)
        return carry

    lax.fori_loop(0, n_main, body, 0)
    for u in range(rem):
        step(2 + 4 * n_main + u, u)

    last_b = (nch - 1) % 2
    last_q = (nch - 1) % 4
    drain(sem_g[last_b], last_b)
    fire_scatters(last_b, last_q)
    drain(sem_s[1 - last_b], 1 - last_b)
    drain(sem_s[last_b], last_b)
    wait_idx(nch % 4)
    wait_idx((nch + 1) % 4)


def _zero_acc(z_hbm, acc, s):
    r0 = s * ROWS_PER_TILE
    pltpu.sync_copy(z_hbm.at[pl.ds(r0, ROWS_PER_TILE)], acc.at[pl.ds(r0, ROWS_PER_TILE)])


def _writeout(acc, out_hbm, s):
    r0 = s * ROWS_PER_TILE
    pltpu.sync_copy(acc.at[pl.ds(r0, ROWS_PER_TILE)], out_hbm.at[pl.ds(r0, ROWS_PER_TILE)])


def _sc_agg1_body(xp, pk_t, pk_i, z8,
                  aggt, aggi,
                  i0, i1, i2, i3, rows0, rows1, acc,
                  si0, si1, si2, si3, sg0, sg1, ss0, ss1):
    c = lax.axis_index("c")
    s = lax.axis_index("s")
    idx = [i0, i1, i2, i3]
    rows = [rows0, rows1]
    sem_i = [si0, si1, si2, si3]
    sem_g = [sg0, sg1]
    sem_s = [ss0, ss1]
    _zero_acc(z8, acc, s)
    plsc.subcore_barrier()

    @pl.when(c == 0)
    def _():
        _edge_stream(pk_t, xp, acc, idx, rows, sem_i, sem_g, sem_s, s, CH1)
        plsc.subcore_barrier()
        _writeout(acc, aggt, s)

    @pl.when(c == 1)
    def _():
        _edge_stream(pk_i, xp, acc, idx, rows, sem_i, sem_g, sem_s, s, CH1)
        plsc.subcore_barrier()
        _writeout(acc, aggi, s)


_sc_agg1 = functools.partial(
    pl.kernel,
    compiler_params=_SC_PARAMS,
    out_type=(
        jax.ShapeDtypeStruct((N_PAD, 8), jnp.float32),
        jax.ShapeDtypeStruct((N_PAD, 8), jnp.float32),
    ),
    mesh=_MESH,
    scratch_types=[
        pltpu.VMEM((CH1 // 64, 128), jnp.int32),
        pltpu.VMEM((CH1 // 64, 128), jnp.int32),
        pltpu.VMEM((CH1 // 64, 128), jnp.int32),
        pltpu.VMEM((CH1 // 64, 128), jnp.int32),
        pltpu.VMEM((CH1, 8), jnp.float32),
        pltpu.VMEM((CH1, 8), jnp.float32),
        pltpu.VMEM_SHARED((N_PAD, 8), jnp.float32),
        pltpu.SemaphoreType.DMA,
        pltpu.SemaphoreType.DMA,
        pltpu.SemaphoreType.DMA,
        pltpu.SemaphoreType.DMA,
        pltpu.SemaphoreType.DMA,
        pltpu.SemaphoreType.DMA,
        pltpu.SemaphoreType.DMA,
        pltpu.SemaphoreType.DMA,
    ],
)(_sc_agg1_body)


def _sc_agg2_body(h1a, h1b, pk_t, pk_i, z16,
                  ota, otb, oia, oib,
                  i0, i1, i2, i3, rows0, rows1, acc,
                  si0, si1, si2, si3, sg0, sg1, ss0, ss1):
    c = lax.axis_index("c")
    s = lax.axis_index("s")
    idx = [i0, i1, i2, i3]
    rows = [rows0, rows1]
    sem_i = [si0, si1, si2, si3]
    sem_g = [sg0, sg1]
    sem_s = [ss0, ss1]

    def task(h, pk, out_e):
        _zero_acc(z16, acc, s)
        plsc.subcore_barrier()
        _edge_stream(pk, h, acc, idx, rows, sem_i, sem_g, sem_s, s, CH2)
        plsc.subcore_barrier()
        _writeout(acc, out_e, s)

    @pl.when(c == 0)
    def _():
        task(h1a, pk_t, ota)
        task(h1a, pk_i, oia)

    @pl.when(c == 1)
    def _():
        task(h1b, pk_t, otb)
        task(h1b, pk_i, oib)


_sc_agg2 = functools.partial(
    pl.kernel,
    compiler_params=_SC_PARAMS,
    out_type=(
        jax.ShapeDtypeStruct((N_PAD, 16), jnp.float32),
        jax.ShapeDtypeStruct((N_PAD, 16), jnp.float32),
        jax.ShapeDtypeStruct((N_PAD, 16), jnp.float32),
        jax.ShapeDtypeStruct((N_PAD, 16), jnp.float32),
    ),
    mesh=_MESH,
    scratch_types=[
        pltpu.VMEM((CH2 // 64, 128), jnp.int32),
        pltpu.VMEM((CH2 // 64, 128), jnp.int32),
        pltpu.VMEM((CH2 // 64, 128), jnp.int32),
        pltpu.VMEM((CH2 // 64, 128), jnp.int32),
        pltpu.VMEM((CH2, 16), jnp.float32),
        pltpu.VMEM((CH2, 16), jnp.float32),
        pltpu.VMEM_SHARED((N_PAD, 16), jnp.float32),
        pltpu.SemaphoreType.DMA,
        pltpu.SemaphoreType.DMA,
        pltpu.SemaphoreType.DMA,
        pltpu.SemaphoreType.DMA,
        pltpu.SemaphoreType.DMA,
        pltpu.SemaphoreType.DMA,
        pltpu.SemaphoreType.DMA,
        pltpu.SemaphoreType.DMA,
    ],
)(_sc_agg2_body)


GRID = (N + 2048 - 1) // 2048


def _tc1_body(aggt_ref, aggi_ref, xp_ref,
              wta_ref, wtb_ref, wia_ref, wib_ref, wra_ref, wrb_ref,
              wsa_ref, wsb_ref, ba_ref, bb_ref, ecnt_ref,
              h1a_ref, h1b_ref):
    f32 = jnp.float32
    aggt = aggt_ref[...]
    aggi = aggi_ref[...]
    xp = xp_ref[...]
    dot = lambda a, b: jnp.dot(a, b, preferred_element_type=f32)
    cntb = dot(aggi, ecnt_ref[...])
    aggim = aggi * (1.0 / jnp.maximum(cntb, 1.0))
    preA = dot(aggt, wta_ref[...]) + dot(aggim, wia_ref[...])         + dot(xp, wra_ref[...]) + ba_ref[...]
    h1a_ref[...] = jnp.maximum(preA, 0.0) + dot(xp, wsa_ref[...])
    preB = dot(aggt, wtb_ref[...]) + dot(aggim, wib_ref[...])         + dot(xp, wrb_ref[...]) + bb_ref[...]
    h1b_ref[...] = jnp.maximum(preB, 0.0) + dot(xp, wsb_ref[...])


def _tc1(aggt_p, aggi_p, xp_p, *weights):
    b8 = pl.BlockSpec((128, 128), lambda i: (i, 0))
    b16 = pl.BlockSpec((128, 256), lambda i: (i, 0))
    cst = lambda shp: pl.BlockSpec(shp, lambda i: (0, 0))
    return pl.pallas_call(
        _tc1_body,
        grid=(GRID,),
        in_specs=[
            b8, b8, b8,
            cst((128, 256)), cst((128, 256)), cst((128, 256)), cst((128, 256)),
            cst((128, 256)), cst((128, 256)), cst((128, 256)), cst((128, 256)),
            cst((1, 256)), cst((1, 256)), cst((128, 128)),
        ],
        out_specs=[b16, b16],
        out_shape=[
            jax.ShapeDtypeStruct((N // 16, 256), jnp.float32),
            jax.ShapeDtypeStruct((N // 16, 256), jnp.float32),
        ],
    )(aggt_p, aggi_p, xp_p, *weights)


def _tc2_body(ta_ref, tb_ref, ia_ref, ib_ref, h1a_ref, h1b_ref, aggi1_ref,
              wt_a, wt_b, wi_a, wi_b, wr_a, wr_b, ws_a, ws_b, b2e_ref,
              wd1e_ref, bd1e_ref, wd2e_ref, bd2e_ref, ecnt16_ref, out_ref):
    f32 = jnp.float32
    dot = lambda a, b: jnp.dot(a, b, preferred_element_type=f32)
    cnt16 = dot(aggi1_ref[...], ecnt16_ref[...])
    rcp16 = 1.0 / jnp.maximum(cnt16, 1.0)
    ia = ia_ref[...] * rcp16
    ib = ib_ref[...] * rcp16
    h1a = h1a_ref[...]
    h1b = h1b_ref[...]
    pre = (
        dot(ta_ref[...], wt_a[...]) + dot(tb_ref[...], wt_b[...])
        + dot(ia, wi_a[...]) + dot(ib, wi_b[...])
        + dot(h1a, wr_a[...]) + dot(h1b, wr_b[...]) + b2e_ref[...]
    )
    h2 = jnp.maximum(pre, 0.0) + dot(h1a, ws_a[...]) + dot(h1b, ws_b[...])
    d = jnp.maximum(dot(h2, wd1e_ref[...]) + bd1e_ref[...], 0.0)
    z = dot(d, wd2e_ref[...]) + bd2e_ref[...]
    out_ref[...] = 1.0 / (1.0 + jnp.exp(-z))


def _tc2(ta_p, tb_p, ia_p, ib_p, h1a_p, h1b_p, aggi1_p, *weights):
    b8 = pl.BlockSpec((128, 128), lambda i: (i, 0))
    b16 = pl.BlockSpec((128, 256), lambda i: (i, 0))
    cst = lambda shp: pl.BlockSpec(shp, lambda i: (0, 0))
    return pl.pallas_call(
        _tc2_body,
        grid=(GRID,),
        in_specs=[
            b16, b16, b16, b16, b16, b16, b8,
            cst((256, 512)), cst((256, 512)), cst((256, 512)), cst((256, 512)),
            cst((256, 512)), cst((256, 512)), cst((256, 512)), cst((256, 512)),
            cst((1, 512)),
            cst((512, 512)), cst((1, 512)), cst((512, 16)), cst((1, 16)),
            cst((128, 256)),
        ],
        out_specs=[pl.BlockSpec((128, 16), lambda i: (i, 0))],
        out_shape=[jax.ShapeDtypeStruct((N // 16, 16), jnp.float32)],
    )(ta_p, tb_p, ia_p, ib_p, h1a_p, h1b_p, aggi1_p, *weights)[0]


def _pack_edges(ei):
    src_p = jnp.concatenate([ei[0], jnp.zeros((E_PAD - E,), jnp.int32)])
    dst_p = jnp.concatenate([ei[1], jnp.full((E_PAD - E,), N, jnp.int32)])
    s3 = src_p.reshape(E_PAD // 512, 4, 128)
    d3 = dst_p.reshape(E_PAD // 512, 4, 128)
    return jnp.concatenate([s3, d3], axis=1).reshape(E_PAD // 64, 128)


def kernel(x_stroke, ei_temp, ei_int,
           Wt1, Wi1, Wr1, Ws1, b1,
           Wt2, Wi2, Wr2, Ws2, b2,
           Wd1, bd1, Wd2, bd2):
    f32 = jnp.float32
    eye16 = jnp.eye(16, dtype=f32)
    kr = lambda w: jnp.kron(eye16, w)

    ones = jnp.ones((N, 1), f32)
    zeros = jnp.zeros((N, 1), f32)
    xp_p = jnp.concatenate([x_stroke, ones, zeros], axis=1).reshape(N // 16, 128)

    pk_t = _pack_edges(ei_temp)
    pk_i = _pack_edges(ei_int)
    z8 = jnp.zeros((N_PAD, 8), f32)
    z16 = jnp.zeros((N_PAD, 16), f32)

    aggt1, aggi1 = _sc_agg1(xp_p.reshape(N, 8), pk_t, pk_i, z8)
    aggt1_p = aggt1.reshape(N_PAD // 16, 128)
    aggi1_p = aggi1.reshape(N_PAD // 16, 128)

    pad2 = lambda w: jnp.pad(w, ((0, 2), (0, 0)))
    Wt1p, Wi1p, Wr1p, Ws1p = pad2(Wt1), pad2(Wi1), pad2(Wr1), pad2(Ws1)
    m8 = jnp.zeros((8, 8), f32).at[6].set(1.0)
    w1 = [kr(Wt1p[:, :16]), kr(Wt1p[:, 16:]), kr(Wi1p[:, :16]), kr(Wi1p[:, 16:]),
          kr(Wr1p[:, :16]), kr(Wr1p[:, 16:]), kr(Ws1p[:, :16]), kr(Ws1p[:, 16:]),
          jnp.tile(b1[:16], 16).reshape(1, 256), jnp.tile(b1[16:], 16).reshape(1, 256),
          kr(m8)]
    h1a_p, h1b_p = _tc1(aggt1_p, aggi1_p, xp_p, *w1)

    ta, tb, ia, ib = _sc_agg2(
        h1a_p.reshape(N, 16), h1b_p.reshape(N, 16), pk_t, pk_i, z16)

    m816 = jnp.zeros((8, 16), f32).at[6].set(1.0)
    w2 = [kr(Wt2[:16]), kr(Wt2[16:]), kr(Wi2[:16]), kr(Wi2[16:]),
          kr(Wr2[:16]), kr(Wr2[16:]), kr(Ws2[:16]), kr(Ws2[16:]),
          jnp.tile(b2, 16).reshape(1, 512),
          kr(Wd1), jnp.tile(bd1, 16).reshape(1, 512),
          kr(Wd2), jnp.tile(bd2, 16).reshape(1, 16),
          kr(m816)]
    out_p = _tc2(
        ta.reshape(N_PAD // 16, 256), tb.reshape(N_PAD // 16, 256),
        ia.reshape(N_PAD // 16, 256), ib.reshape(N_PAD // 16, 256),
        h1a_p, h1b_p, aggi1_p, *w2)
    return out_p.reshape(N, 1)
